# baseline (speedup 1.0000x reference)
import dataclasses
import functools

import jax
import jax.numpy as jnp
from jax import lax
from jax.experimental import pallas as pl
from jax.experimental.pallas import tpu as pltpu
from jax.experimental.pallas import tpu_sc as plsc

N = 10000
E = 160000
C = 128
L = 2
HEADS = 8
DH = C // HEADS
GROUPS = 8

NC = 2
NS = 16
NW = NC * NS
NPAD = 10112
SEG = NPAD // NS
CHUNK = 32
EPT = 5120
EPAD = NW * EPT
NCHUNK = EPT // CHUNK
IBLK = 8
NBLK = NCHUNK // IBLK

_mesh = plsc.VectorSubcoreMesh(core_axis_name="c", subcore_axis_name="s")
_cp = pltpu.CompilerParams()
if "needs_layout_passes" in pltpu.CompilerParams.__dataclass_fields__:
    _cp = dataclasses.replace(_cp, needs_layout_passes=False)

DEGC = 128
DEGN = EPT // DEGC


@functools.partial(
    pl.kernel,
    out_type=jax.ShapeDtypeStruct((NC, NS, SEG), jnp.float32),
    mesh=_mesh,
    compiler_params=_cp,
    scratch_types=[
        pltpu.VMEM((DEGC,), jnp.float32),
        pltpu.VMEM((SEG,), jnp.float32),
        pltpu.VMEM((DEGN, DEGC), jnp.int32),
        pltpu.VMEM_SHARED((NPAD,), jnp.float32),
    ],
)
def _deg_kernel(col_hbm, deg_hbm, ones, acc, idx, sh):
    cid = lax.axis_index("c")
    sid = lax.axis_index("s")
    wid = sid * NC + cid
    one16 = jnp.full((16,), 1.0, jnp.float32)
    zero16 = jnp.zeros((16,), jnp.float32)

    @pl.loop(0, DEGC, step=16)
    def _(i):
        ones[pl.ds(i, 16)] = one16

    @pl.loop(0, SEG, step=16)
    def _(i):
        acc[pl.ds(i, 16)] = zero16

    base = pl.multiple_of(sid * SEG, 8)
    pltpu.sync_copy(acc, sh.at[pl.ds(base, SEG)])
    plsc.subcore_barrier()

    pltpu.sync_copy(col_hbm.at[wid], idx)

    @pl.loop(0, DEGN)
    def _(t):
        pltpu.sync_copy(ones, sh.at[idx.at[t]], add=True)

    plsc.subcore_barrier()
    pltpu.sync_copy(sh.at[pl.ds(base, SEG)], acc)
    pltpu.sync_copy(acc, deg_hbm.at[cid, sid])


_BLK2 = NPAD // 8


def _tables_body(x_ref, w2k_ref, w2v_ref, wq_ref, bk2_ref, bv2_ref, bq_ref,
                 degp_ref, kv_ref, q_ref, dis_ref):
    deg = degp_ref[:, 0:1] + degp_ref[:, 1:2] + 1.0
    dis = lax.rsqrt(deg)
    x = x_ref[...]
    k = jnp.dot(x, w2k_ref[...], preferred_element_type=jnp.float32)
    k = k + bk2_ref[...]
    v = jnp.dot(x, w2v_ref[...], preferred_element_type=jnp.float32)
    v = (v + bv2_ref[...]) * dis
    q = jnp.dot(x[:, C:], wq_ref[...], preferred_element_type=jnp.float32)
    q_ref[...] = q + bq_ref[...]
    kv_ref[...] = jnp.concatenate([k, v], axis=1)
    dis_ref[...] = dis


def _tables_call(xp, w2k, w2v, wq1, bk2, bv2, bq1, degp2):
    nblk = NPAD // _BLK2
    return pl.pallas_call(
        _tables_body,
        grid=(nblk,),
        in_specs=[
            pl.BlockSpec((_BLK2, 2 * C), lambda i: (i, 0)),
            pl.BlockSpec((2 * C, 2 * C), lambda i: (0, 0)),
            pl.BlockSpec((2 * C, 2 * C), lambda i: (0, 0)),
            pl.BlockSpec((C, C), lambda i: (0, 0)),
            pl.BlockSpec((1, 2 * C), lambda i: (0, 0)),
            pl.BlockSpec((1, 2 * C), lambda i: (0, 0)),
            pl.BlockSpec((1, C), lambda i: (0, 0)),
            pl.BlockSpec((_BLK2, 2), lambda i: (i, 0)),
        ],
        out_specs=[
            pl.BlockSpec((_BLK2, 4 * C), lambda i: (i, 0)),
            pl.BlockSpec((_BLK2, C), lambda i: (i, 0)),
            pl.BlockSpec((_BLK2, 1), lambda i: (i, 0)),
        ],
        out_shape=[
            jax.ShapeDtypeStruct((NPAD, 4 * C), jnp.float32),
            jax.ShapeDtypeStruct((NPAD, C), jnp.float32),
            jax.ShapeDtypeStruct((NPAD, 1), jnp.float32),
        ],
    )(xp, w2k, w2v, wq1, bk2, bv2, bq1, degp2)


@functools.partial(
    pl.kernel,
    out_type=jax.ShapeDtypeStruct((NC, NS, SEG, C), jnp.float32),
    mesh=_mesh,
    compiler_params=_cp,
    scratch_types=[
        pltpu.VMEM((2 * IBLK, CHUNK), jnp.int32),
        pltpu.VMEM((CHUNK, 4 * C), jnp.float32),
        pltpu.VMEM((CHUNK, 4 * C), jnp.float32),
        pltpu.VMEM((CHUNK, C), jnp.float32),
        pltpu.VMEM((CHUNK, C), jnp.float32),
        pltpu.VMEM((CHUNK, C), jnp.float32),
        pltpu.VMEM_SHARED((NPAD, C), jnp.float32),
        pltpu.SemaphoreType.DMA,
        pltpu.SemaphoreType.DMA,
        pltpu.SemaphoreType.DMA,
    ],
)
def _edge_kernel(rc_hbm, kv_hbm, q_hbm, agg_hbm,
                 idxb, kv_a, kv_b, q_a, q_b, msg,
                 aggsh, sem_a, sem_b, sem_s):
    cid = lax.axis_index("c")
    sid = lax.axis_index("s")
    wid = sid * NC + cid
    zeros = jnp.zeros((16,), jnp.float32)

    @pl.loop(0, CHUNK)
    def _(r):
        for cc in range(0, C, 16):
            msg[r, pl.ds(cc, 16)] = zeros

    base = pl.multiple_of(sid * SEG, 8)
    nfull = SEG // CHUNK
    for t in range(nfull):
        pltpu.sync_copy(msg, aggsh.at[pl.ds(base + t * CHUNK, CHUNK)])
    rem = SEG - nfull * CHUNK
    if rem:
        pltpu.sync_copy(msg.at[pl.ds(0, rem)],
                        aggsh.at[pl.ds(base + nfull * CHUNK, rem)])
    plsc.subcore_barrier()

    kvb = (kv_a, kv_b)
    qb = (q_a, q_b)
    gsem = (sem_a, sem_b)

    def _issue(p):
        pltpu.make_async_copy(kv_hbm.at[idxb.at[p]], kvb[p % 2],
                              gsem[p % 2]).start()
        pltpu.make_async_copy(q_hbm.at[idxb.at[IBLK + p]], qb[p % 2],
                              gsem[p % 2]).start()

    def _wait(p):
        pltpu.make_async_copy(kv_hbm.at[idxb.at[p]], kvb[p % 2],
                              gsem[p % 2]).wait()
        pltpu.make_async_copy(q_hbm.at[idxb.at[IBLK + p]], qb[p % 2],
                              gsem[p % 2]).wait()

    @pl.loop(0, NBLK)
    def _(b):
        pltpu.sync_copy(rc_hbm.at[wid, b], idxb)
        _issue(0)
        for p in range(IBLK):
            if p + 1 < IBLK:
                _issue(p + 1)
            _wait(p)
            kvp, qp = kvb[p % 2], qb[p % 2]

            lanes = lax.iota(jnp.int32, 16)

            def _cf(v):
                return jnp.full((16,), v, jnp.int32)

            for g in range(CHUNK // 16):
                rows = lanes + g * 16

                @pl.loop(0, HEADS)
                def _(h):
                    o = h * DH
                    cols = jnp.full((16,), o, jnp.int32)
                    szero = jnp.zeros((16,), jnp.float32)

                    @pl.loop(0, DH, init_carry=(szero, szero), unroll=4)
                    def dots(d, carry):
                        s0c, s1c = carry
                        qv = plsc.load_gather(qp, [rows, cols + d])
                        k0v = plsc.load_gather(kvp, [rows, cols + d])
                        k1v = plsc.load_gather(kvp, [rows, cols + (C + d)])
                        return s0c + qv * k0v, s1c + qv * k1v

                    s0, s1 = dots
                    s0 = s0 * 0.25
                    s1 = s1 * 0.25
                    m = jnp.maximum(jnp.maximum(s0, s1), 0.0)
                    e0 = jnp.exp(s0 - m)
                    e1 = jnp.exp(s1 - m)
                    r = 1.0 / (e0 + e1 + jnp.exp(-m))
                    a0 = e0 * r
                    a1 = e1 * r
                    @pl.loop(0, DH, unroll=4)
                    def _(d):
                        v0v = plsc.load_gather(kvp,
                                               [rows, cols + (2 * C + d)])
                        v1v = plsc.load_gather(kvp,
                                               [rows, cols + (3 * C + d)])
                        plsc.store_scatter(msg, [rows, cols + d],
                                           a0 * v0v + a1 * v1v)

            pltpu.sync_copy(msg, aggsh.at[idxb.at[IBLK + p]], add=True)

    plsc.subcore_barrier()
    pltpu.sync_copy(aggsh.at[pl.ds(base, SEG)], agg_hbm.at[cid, sid])


_BLK4 = NPAD // 8


def _combine_body(agg_ref, q_ref, kv_ref, dis_ref, g_ref, gt_ref,
                  h_ref, stats_ref):
    i = pl.program_id(0)
    agg = agg_ref[0] + agg_ref[1]
    q = q_ref[...]
    k0 = kv_ref[:, 0:C]
    k1 = kv_ref[:, C:2 * C]
    v0 = kv_ref[:, 2 * C:3 * C]
    v1 = kv_ref[:, 3 * C:4 * C]
    gmat = g_ref[...]
    gtmat = gt_ref[...]
    s0 = jnp.dot(q * k0, gmat, preferred_element_type=jnp.float32) * 0.25
    s1 = jnp.dot(q * k1, gmat, preferred_element_type=jnp.float32) * 0.25
    m = jnp.maximum(jnp.maximum(s0, s1), 0.0)
    e0 = jnp.exp(s0 - m)
    e1 = jnp.exp(s1 - m)
    r = 1.0 / (e0 + e1 + jnp.exp(-m))
    a0 = jnp.dot(e0 * r, gtmat, preferred_element_type=jnp.float32)
    a1 = jnp.dot(e1 * r, gtmat, preferred_element_type=jnp.float32)
    self_out = a0 * v0 + a1 * v1
    h = jnp.maximum(dis_ref[...] * (agg + self_out), 0.0)
    h_ref[...] = h
    rid = lax.broadcasted_iota(jnp.int32, (_BLK4, 1), 0) + i * _BLK4
    hm = jnp.where(rid < N, h, 0.0)

    @pl.when(i == 0)
    def _():
        stats_ref[...] = jnp.zeros_like(stats_ref)

    stats_ref[0:1, :] += jnp.sum(hm, axis=0, keepdims=True)
    stats_ref[1:2, :] += jnp.sum(hm * hm, axis=0, keepdims=True)


def _combine_call(aggp, q, kv, dis, gmat, gtmat):
    nblk = NPAD // _BLK4
    return pl.pallas_call(
        _combine_body,
        grid=(nblk,),
        in_specs=[
            pl.BlockSpec((2, _BLK4, C), lambda i: (0, i, 0)),
            pl.BlockSpec((_BLK4, C), lambda i: (i, 0)),
            pl.BlockSpec((_BLK4, 4 * C), lambda i: (i, 0)),
            pl.BlockSpec((_BLK4, 1), lambda i: (i, 0)),
            pl.BlockSpec((C, HEADS), lambda i: (0, 0)),
            pl.BlockSpec((HEADS, C), lambda i: (0, 0)),
        ],
        out_specs=[
            pl.BlockSpec((_BLK4, C), lambda i: (i, 0)),
            pl.BlockSpec((2, C), lambda i: (0, 0)),
        ],
        out_shape=[
            jax.ShapeDtypeStruct((NPAD, C), jnp.float32),
            jax.ShapeDtypeStruct((2, C), jnp.float32),
        ],
    )(aggp, q, kv, dis, gmat, gtmat)


def _bn_body(h_ref, stats_ref, gamma_ref, beta_ref, out_ref):
    inv_n = 1.0 / N
    mean = stats_ref[0:1, :] * inv_n
    var = stats_ref[1:2, :] * inv_n - mean * mean
    scale = lax.rsqrt(var + 1e-5) * gamma_ref[...]
    out_ref[...] = (h_ref[...] - mean) * scale + beta_ref[...]


def _bn_call(h, stats, gamma, beta):
    return pl.pallas_call(
        _bn_body,
        out_shape=jax.ShapeDtypeStruct((N, C), jnp.float32),
    )(h, stats, gamma, beta)


def _block_diag8(w):
    cg = C // GROUPS
    cols = []
    for g in range(GROUPS):
        col = [jnp.zeros((cg, cg), w.dtype)] * GROUPS
        col[g] = w[g]
        cols.append(jnp.concatenate(col, axis=0))
    return jnp.concatenate(cols, axis=1)


def kernel(all_node_features, edge_indices, wq, bq, wk, bk, wv, bv,
           gamma, beta):
    x = all_node_features
    row = edge_indices[0]
    col = edge_indices[1]

    pad = EPAD - E
    rowp = jnp.concatenate([row, jnp.zeros((pad,), row.dtype)])
    colp = jnp.concatenate([col, jnp.full((pad,), N, col.dtype)])
    row4 = rowp.reshape(NW, NBLK, IBLK, CHUNK)
    col4 = colp.reshape(NW, NBLK, IBLK, CHUNK)
    rc4 = jnp.concatenate([row4, col4], axis=2)
    col2 = colp.reshape(NW, DEGN, DEGC)

    wk1 = _block_diag8(wk)
    wv1 = _block_diag8(wv)
    wq1 = _block_diag8(wq)
    z = jnp.zeros((C, C), jnp.float32)
    w2k = jnp.concatenate(
        [jnp.concatenate([wk1, z], axis=1), jnp.concatenate([z, wk1], axis=1)],
        axis=0)
    w2v = jnp.concatenate(
        [jnp.concatenate([wv1, z], axis=1), jnp.concatenate([z, wv1], axis=1)],
        axis=0)
    bk2 = jnp.tile(bk, 2)[None, :]
    bv2 = jnp.tile(bv, 2)[None, :]
    bq1 = bq[None, :]

    eyeh = jnp.eye(HEADS, dtype=jnp.float32)
    gmat = jnp.repeat(eyeh, DH, axis=0)
    gtmat = jnp.repeat(eyeh, DH, axis=1)

    xp = jnp.pad(x.reshape(N, L * C), ((0, NPAD - N), (0, 0)))

    degp = _deg_kernel(col2)
    degp2 = degp.reshape(NC, NPAD).T
    kv, q, dis = _tables_call(xp, w2k, w2v, wq1, bk2, bv2, bq1, degp2)
    aggp = _edge_kernel(rc4, kv, q)
    aggp = aggp.reshape(NC, NPAD, C)
    h, stats = _combine_call(aggp, q, kv, dis, gmat, gtmat)
    return _bn_call(h[:N], stats, gamma[None, :], beta[None, :])


# speedup vs baseline: 1.4003x; 1.4003x over previous
import dataclasses
import functools

import jax
import jax.numpy as jnp
from jax import lax
from jax.experimental import pallas as pl
from jax.experimental.pallas import tpu as pltpu
from jax.experimental.pallas import tpu_sc as plsc

N = 10000
E = 160000
C = 128
L = 2
HEADS = 8
DH = C // HEADS
GROUPS = 8

NC = 2
NS = 16
NW = NC * NS
NPAD = 10112
SEG = NPAD // NS
CHUNK = 32
EPT = 5120
EPAD = NW * EPT
NCHUNK = EPT // CHUNK
IBLK = 8
NBLK = NCHUNK // IBLK

_mesh = plsc.VectorSubcoreMesh(core_axis_name="c", subcore_axis_name="s")
_cp = pltpu.CompilerParams()
if "needs_layout_passes" in pltpu.CompilerParams.__dataclass_fields__:
    _cp = dataclasses.replace(_cp, needs_layout_passes=False)

DEGC = 128
DEGN = EPT // DEGC


@functools.partial(
    pl.kernel,
    out_type=jax.ShapeDtypeStruct((NC, NS, SEG), jnp.float32),
    mesh=_mesh,
    compiler_params=_cp,
    scratch_types=[
        pltpu.VMEM((DEGC,), jnp.float32),
        pltpu.VMEM((SEG,), jnp.float32),
        pltpu.VMEM((DEGN, DEGC), jnp.int32),
        pltpu.VMEM_SHARED((NPAD,), jnp.float32),
    ],
)
def _deg_kernel(col_hbm, deg_hbm, ones, acc, idx, sh):
    cid = lax.axis_index("c")
    sid = lax.axis_index("s")
    wid = sid * NC + cid
    one16 = jnp.full((16,), 1.0, jnp.float32)
    zero16 = jnp.zeros((16,), jnp.float32)

    @pl.loop(0, DEGC, step=16)
    def _(i):
        ones[pl.ds(i, 16)] = one16

    @pl.loop(0, SEG, step=16)
    def _(i):
        acc[pl.ds(i, 16)] = zero16

    base = pl.multiple_of(sid * SEG, 8)
    pltpu.sync_copy(acc, sh.at[pl.ds(base, SEG)])
    plsc.subcore_barrier()

    pltpu.sync_copy(col_hbm.at[wid], idx)

    @pl.loop(0, DEGN)
    def _(t):
        pltpu.sync_copy(ones, sh.at[idx.at[t]], add=True)

    plsc.subcore_barrier()
    pltpu.sync_copy(sh.at[pl.ds(base, SEG)], acc)
    pltpu.sync_copy(acc, deg_hbm.at[cid, sid])


_BLK2 = NPAD // 8


def _tables_body(x_ref, w2k_ref, w2v_ref, wq_ref, bk2_ref, bv2_ref, bq_ref,
                 degp_ref, kv_ref, q_ref, dis_ref):
    deg = degp_ref[:, 0:1] + degp_ref[:, 1:2] + 1.0
    dis = lax.rsqrt(deg)
    x = x_ref[...]
    k = jnp.dot(x, w2k_ref[...], preferred_element_type=jnp.float32)
    k = k + bk2_ref[...]
    v = jnp.dot(x, w2v_ref[...], preferred_element_type=jnp.float32)
    v = (v + bv2_ref[...]) * dis
    q = jnp.dot(x[:, C:], wq_ref[...], preferred_element_type=jnp.float32)
    q_ref[...] = q + bq_ref[...]
    kv_ref[...] = jnp.concatenate([k, v], axis=1)
    dis_ref[...] = dis


def _tables_call(xp, w2k, w2v, wq1, bk2, bv2, bq1, degp2):
    nblk = NPAD // _BLK2
    return pl.pallas_call(
        _tables_body,
        grid=(nblk,),
        in_specs=[
            pl.BlockSpec((_BLK2, 2 * C), lambda i: (i, 0)),
            pl.BlockSpec((2 * C, 2 * C), lambda i: (0, 0)),
            pl.BlockSpec((2 * C, 2 * C), lambda i: (0, 0)),
            pl.BlockSpec((C, C), lambda i: (0, 0)),
            pl.BlockSpec((1, 2 * C), lambda i: (0, 0)),
            pl.BlockSpec((1, 2 * C), lambda i: (0, 0)),
            pl.BlockSpec((1, C), lambda i: (0, 0)),
            pl.BlockSpec((_BLK2, 2), lambda i: (i, 0)),
        ],
        out_specs=[
            pl.BlockSpec((_BLK2, 4 * C), lambda i: (i, 0)),
            pl.BlockSpec((_BLK2, C), lambda i: (i, 0)),
            pl.BlockSpec((_BLK2, 1), lambda i: (i, 0)),
        ],
        out_shape=[
            jax.ShapeDtypeStruct((NPAD, 4 * C), jnp.float32),
            jax.ShapeDtypeStruct((NPAD, C), jnp.float32),
            jax.ShapeDtypeStruct((NPAD, 1), jnp.float32),
        ],
    )(xp, w2k, w2v, wq1, bk2, bv2, bq1, degp2)


@functools.partial(
    pl.kernel,
    out_type=jax.ShapeDtypeStruct((NC, NS, SEG, C), jnp.float32),
    mesh=_mesh,
    compiler_params=_cp,
    scratch_types=[
        pltpu.VMEM((2 * IBLK, CHUNK), jnp.int32),
        pltpu.VMEM((CHUNK, 4 * C), jnp.float32),
        pltpu.VMEM((CHUNK, 4 * C), jnp.float32),
        pltpu.VMEM((CHUNK, C), jnp.float32),
        pltpu.VMEM((CHUNK, C), jnp.float32),
        pltpu.VMEM((CHUNK, C), jnp.float32),
        pltpu.VMEM_SHARED((NPAD, C), jnp.float32),
        pltpu.SemaphoreType.DMA,
        pltpu.SemaphoreType.DMA,
        pltpu.SemaphoreType.DMA,
    ],
)
def _edge_kernel(rc_hbm, kv_hbm, q_hbm, agg_hbm,
                 idxb, kv_a, kv_b, q_a, q_b, msg,
                 aggsh, sem_a, sem_b, sem_s):
    cid = lax.axis_index("c")
    sid = lax.axis_index("s")
    wid = sid * NC + cid
    zeros = jnp.zeros((16,), jnp.float32)

    @pl.loop(0, CHUNK)
    def _(r):
        for cc in range(0, C, 16):
            msg[r, pl.ds(cc, 16)] = zeros

    base = pl.multiple_of(sid * SEG, 8)
    nfull = SEG // CHUNK
    for t in range(nfull):
        pltpu.sync_copy(msg, aggsh.at[pl.ds(base + t * CHUNK, CHUNK)])
    rem = SEG - nfull * CHUNK
    if rem:
        pltpu.sync_copy(msg.at[pl.ds(0, rem)],
                        aggsh.at[pl.ds(base + nfull * CHUNK, rem)])
    plsc.subcore_barrier()

    kvb = (kv_a, kv_b)
    qb = (q_a, q_b)
    gsem = (sem_a, sem_b)

    def _issue(p):
        pltpu.make_async_copy(kv_hbm.at[idxb.at[p]], kvb[p % 2],
                              gsem[p % 2]).start()
        pltpu.make_async_copy(q_hbm.at[idxb.at[IBLK + p]], qb[p % 2],
                              gsem[p % 2]).start()

    def _wait(p):
        pltpu.make_async_copy(kv_hbm.at[idxb.at[p]], kvb[p % 2],
                              gsem[p % 2]).wait()
        pltpu.make_async_copy(q_hbm.at[idxb.at[IBLK + p]], qb[p % 2],
                              gsem[p % 2]).wait()

    @pl.loop(0, NBLK)
    def _(b):
        pltpu.sync_copy(rc_hbm.at[wid, b], idxb)
        _issue(0)
        for p in range(IBLK):
            if p + 1 < IBLK:
                _issue(p + 1)
            _wait(p)
            kvp, qp = kvb[p % 2], qb[p % 2]

            lanes = lax.iota(jnp.int32, 16)
            perms = [jnp.bitwise_xor(lanes, k) for k in (8, 4, 2, 1)]

            def lsum(v):
                for pm in perms:
                    v = v + v.at[pm].get(mode="promise_in_bounds")
                return v

            @pl.loop(0, CHUNK)
            def _(e):
                for h in range(HEADS):
                    o = h * DH
                    q = qp[e, pl.ds(o, DH)]
                    k0 = kvp[e, pl.ds(o, DH)]
                    k1 = kvp[e, pl.ds(C + o, DH)]
                    s0 = lsum(q * k0) * 0.25
                    s1 = lsum(q * k1) * 0.25
                    m = jnp.maximum(jnp.maximum(s0, s1), 0.0)
                    e0 = jnp.exp(s0 - m)
                    e1 = jnp.exp(s1 - m)
                    r = 1.0 / (e0 + e1 + jnp.exp(-m))
                    v0 = kvp[e, pl.ds(2 * C + o, DH)]
                    v1 = kvp[e, pl.ds(3 * C + o, DH)]
                    msg[e, pl.ds(o, DH)] = (e0 * r) * v0 + (e1 * r) * v1

            pltpu.sync_copy(msg, aggsh.at[idxb.at[IBLK + p]], add=True)

    plsc.subcore_barrier()
    pltpu.sync_copy(aggsh.at[pl.ds(base, SEG)], agg_hbm.at[cid, sid])


_BLK4 = NPAD // 8


def _combine_body(agg_ref, q_ref, kv_ref, dis_ref, g_ref, gt_ref,
                  h_ref, stats_ref):
    i = pl.program_id(0)
    agg = agg_ref[0] + agg_ref[1]
    q = q_ref[...]
    k0 = kv_ref[:, 0:C]
    k1 = kv_ref[:, C:2 * C]
    v0 = kv_ref[:, 2 * C:3 * C]
    v1 = kv_ref[:, 3 * C:4 * C]
    gmat = g_ref[...]
    gtmat = gt_ref[...]
    s0 = jnp.dot(q * k0, gmat, preferred_element_type=jnp.float32) * 0.25
    s1 = jnp.dot(q * k1, gmat, preferred_element_type=jnp.float32) * 0.25
    m = jnp.maximum(jnp.maximum(s0, s1), 0.0)
    e0 = jnp.exp(s0 - m)
    e1 = jnp.exp(s1 - m)
    r = 1.0 / (e0 + e1 + jnp.exp(-m))
    a0 = jnp.dot(e0 * r, gtmat, preferred_element_type=jnp.float32)
    a1 = jnp.dot(e1 * r, gtmat, preferred_element_type=jnp.float32)
    self_out = a0 * v0 + a1 * v1
    h = jnp.maximum(dis_ref[...] * (agg + self_out), 0.0)
    h_ref[...] = h
    rid = lax.broadcasted_iota(jnp.int32, (_BLK4, 1), 0) + i * _BLK4
    hm = jnp.where(rid < N, h, 0.0)

    @pl.when(i == 0)
    def _():
        stats_ref[...] = jnp.zeros_like(stats_ref)

    stats_ref[0:1, :] += jnp.sum(hm, axis=0, keepdims=True)
    stats_ref[1:2, :] += jnp.sum(hm * hm, axis=0, keepdims=True)


def _combine_call(aggp, q, kv, dis, gmat, gtmat):
    nblk = NPAD // _BLK4
    return pl.pallas_call(
        _combine_body,
        grid=(nblk,),
        in_specs=[
            pl.BlockSpec((2, _BLK4, C), lambda i: (0, i, 0)),
            pl.BlockSpec((_BLK4, C), lambda i: (i, 0)),
            pl.BlockSpec((_BLK4, 4 * C), lambda i: (i, 0)),
            pl.BlockSpec((_BLK4, 1), lambda i: (i, 0)),
            pl.BlockSpec((C, HEADS), lambda i: (0, 0)),
            pl.BlockSpec((HEADS, C), lambda i: (0, 0)),
        ],
        out_specs=[
            pl.BlockSpec((_BLK4, C), lambda i: (i, 0)),
            pl.BlockSpec((2, C), lambda i: (0, 0)),
        ],
        out_shape=[
            jax.ShapeDtypeStruct((NPAD, C), jnp.float32),
            jax.ShapeDtypeStruct((2, C), jnp.float32),
        ],
    )(aggp, q, kv, dis, gmat, gtmat)


def _bn_body(h_ref, stats_ref, gamma_ref, beta_ref, out_ref):
    inv_n = 1.0 / N
    mean = stats_ref[0:1, :] * inv_n
    var = stats_ref[1:2, :] * inv_n - mean * mean
    scale = lax.rsqrt(var + 1e-5) * gamma_ref[...]
    out_ref[...] = (h_ref[...] - mean) * scale + beta_ref[...]


def _bn_call(h, stats, gamma, beta):
    return pl.pallas_call(
        _bn_body,
        out_shape=jax.ShapeDtypeStruct((N, C), jnp.float32),
    )(h, stats, gamma, beta)


def _block_diag8(w):
    cg = C // GROUPS
    cols = []
    for g in range(GROUPS):
        col = [jnp.zeros((cg, cg), w.dtype)] * GROUPS
        col[g] = w[g]
        cols.append(jnp.concatenate(col, axis=0))
    return jnp.concatenate(cols, axis=1)


def kernel(all_node_features, edge_indices, wq, bq, wk, bk, wv, bv,
           gamma, beta):
    x = all_node_features
    row = edge_indices[0]
    col = edge_indices[1]

    pad = EPAD - E
    rowp = jnp.concatenate([row, jnp.zeros((pad,), row.dtype)])
    colp = jnp.concatenate([col, jnp.full((pad,), N, col.dtype)])
    row4 = rowp.reshape(NW, NBLK, IBLK, CHUNK)
    col4 = colp.reshape(NW, NBLK, IBLK, CHUNK)
    rc4 = jnp.concatenate([row4, col4], axis=2)
    col2 = colp.reshape(NW, DEGN, DEGC)

    wk1 = _block_diag8(wk)
    wv1 = _block_diag8(wv)
    wq1 = _block_diag8(wq)
    z = jnp.zeros((C, C), jnp.float32)
    w2k = jnp.concatenate(
        [jnp.concatenate([wk1, z], axis=1), jnp.concatenate([z, wk1], axis=1)],
        axis=0)
    w2v = jnp.concatenate(
        [jnp.concatenate([wv1, z], axis=1), jnp.concatenate([z, wv1], axis=1)],
        axis=0)
    bk2 = jnp.tile(bk, 2)[None, :]
    bv2 = jnp.tile(bv, 2)[None, :]
    bq1 = bq[None, :]

    eyeh = jnp.eye(HEADS, dtype=jnp.float32)
    gmat = jnp.repeat(eyeh, DH, axis=0)
    gtmat = jnp.repeat(eyeh, DH, axis=1)

    xp = jnp.pad(x.reshape(N, L * C), ((0, NPAD - N), (0, 0)))

    degp = _deg_kernel(col2)
    degp2 = degp.reshape(NC, NPAD).T
    kv, q, dis = _tables_call(xp, w2k, w2v, wq1, bk2, bv2, bq1, degp2)
    aggp = _edge_kernel(rc4, kv, q)
    aggp = aggp.reshape(NC, NPAD, C)
    h, stats = _combine_call(aggp, q, kv, dis, gmat, gtmat)
    return _bn_call(h[:N], stats, gamma[None, :], beta[None, :])


# speedup vs baseline: 1.4349x; 1.0247x over previous
import dataclasses
import functools

import jax
import jax.numpy as jnp
from jax import lax
from jax.experimental import pallas as pl
from jax.experimental.pallas import tpu as pltpu
from jax.experimental.pallas import tpu_sc as plsc

N = 10000
E = 160000
C = 128
L = 2
HEADS = 8
DH = C // HEADS
GROUPS = 8

NC = 2
NS = 16
NW = NC * NS
NPAD = 10112
SEG = NPAD // NS
CHUNK = 32
EPT = 5120
EPAD = NW * EPT
NCHUNK = EPT // CHUNK
IBLK = 8
NBLK = NCHUNK // IBLK

_mesh = plsc.VectorSubcoreMesh(core_axis_name="c", subcore_axis_name="s")
_cp = pltpu.CompilerParams()
if "needs_layout_passes" in pltpu.CompilerParams.__dataclass_fields__:
    _cp = dataclasses.replace(_cp, needs_layout_passes=False)

DEGC = 128
DEGN = EPT // DEGC


@functools.partial(
    pl.kernel,
    out_type=jax.ShapeDtypeStruct((NC, NS, SEG), jnp.float32),
    mesh=_mesh,
    compiler_params=_cp,
    scratch_types=[
        pltpu.VMEM((DEGC,), jnp.float32),
        pltpu.VMEM((SEG,), jnp.float32),
        pltpu.VMEM((DEGN, DEGC), jnp.int32),
        pltpu.VMEM_SHARED((NPAD,), jnp.float32),
    ],
)
def _deg_kernel(col_hbm, deg_hbm, ones, acc, idx, sh):
    cid = lax.axis_index("c")
    sid = lax.axis_index("s")
    wid = sid * NC + cid
    one16 = jnp.full((16,), 1.0, jnp.float32)
    zero16 = jnp.zeros((16,), jnp.float32)

    @pl.loop(0, DEGC, step=16)
    def _(i):
        ones[pl.ds(i, 16)] = one16

    @pl.loop(0, SEG, step=16)
    def _(i):
        acc[pl.ds(i, 16)] = zero16

    base = pl.multiple_of(sid * SEG, 8)
    pltpu.sync_copy(acc, sh.at[pl.ds(base, SEG)])
    plsc.subcore_barrier()

    pltpu.sync_copy(col_hbm.at[wid], idx)

    @pl.loop(0, DEGN)
    def _(t):
        pltpu.sync_copy(ones, sh.at[idx.at[t]], add=True)

    plsc.subcore_barrier()
    pltpu.sync_copy(sh.at[pl.ds(base, SEG)], acc)
    pltpu.sync_copy(acc, deg_hbm.at[cid, sid])


_BLK2 = NPAD // 8


def _tables_body(x_ref, w2k_ref, w2v_ref, wq_ref, bk2_ref, bv2_ref, bq_ref,
                 degp_ref, kv_ref, q_ref, dis_ref):
    deg = degp_ref[:, 0:1] + degp_ref[:, 1:2] + 1.0
    dis = lax.rsqrt(deg)
    x = x_ref[...]
    k = jnp.dot(x, w2k_ref[...], preferred_element_type=jnp.float32)
    k = k + bk2_ref[...]
    v = jnp.dot(x, w2v_ref[...], preferred_element_type=jnp.float32)
    v = (v + bv2_ref[...]) * dis
    q = jnp.dot(x[:, C:], wq_ref[...], preferred_element_type=jnp.float32)
    q_ref[...] = (q + bq_ref[...]) * 0.25
    kv_ref[...] = jnp.concatenate([k, v], axis=1)
    dis_ref[...] = dis


def _tables_call(xp, w2k, w2v, wq1, bk2, bv2, bq1, degp2):
    nblk = NPAD // _BLK2
    return pl.pallas_call(
        _tables_body,
        grid=(nblk,),
        in_specs=[
            pl.BlockSpec((_BLK2, 2 * C), lambda i: (i, 0)),
            pl.BlockSpec((2 * C, 2 * C), lambda i: (0, 0)),
            pl.BlockSpec((2 * C, 2 * C), lambda i: (0, 0)),
            pl.BlockSpec((C, C), lambda i: (0, 0)),
            pl.BlockSpec((1, 2 * C), lambda i: (0, 0)),
            pl.BlockSpec((1, 2 * C), lambda i: (0, 0)),
            pl.BlockSpec((1, C), lambda i: (0, 0)),
            pl.BlockSpec((_BLK2, 2), lambda i: (i, 0)),
        ],
        out_specs=[
            pl.BlockSpec((_BLK2, 4 * C), lambda i: (i, 0)),
            pl.BlockSpec((_BLK2, C), lambda i: (i, 0)),
            pl.BlockSpec((_BLK2, 1), lambda i: (i, 0)),
        ],
        out_shape=[
            jax.ShapeDtypeStruct((NPAD, 4 * C), jnp.float32),
            jax.ShapeDtypeStruct((NPAD, C), jnp.float32),
            jax.ShapeDtypeStruct((NPAD, 1), jnp.float32),
        ],
    )(xp, w2k, w2v, wq1, bk2, bv2, bq1, degp2)


@functools.partial(
    pl.kernel,
    out_type=jax.ShapeDtypeStruct((NC, NS, SEG, C), jnp.float32),
    mesh=_mesh,
    compiler_params=_cp,
    scratch_types=[
        pltpu.VMEM((2 * IBLK, CHUNK), jnp.int32),
        pltpu.VMEM((CHUNK, 4 * C), jnp.float32),
        pltpu.VMEM((CHUNK, 4 * C), jnp.float32),
        pltpu.VMEM((CHUNK, C), jnp.float32),
        pltpu.VMEM((CHUNK, C), jnp.float32),
        pltpu.VMEM((CHUNK, C), jnp.float32),
        pltpu.VMEM_SHARED((NPAD, C), jnp.float32),
        pltpu.SemaphoreType.DMA,
        pltpu.SemaphoreType.DMA,
        pltpu.SemaphoreType.DMA,
    ],
)
def _edge_kernel(rc_hbm, kv_hbm, q_hbm, agg_hbm,
                 idxb, kv_a, kv_b, q_a, q_b, msg,
                 aggsh, sem_a, sem_b, sem_s):
    cid = lax.axis_index("c")
    sid = lax.axis_index("s")
    wid = sid * NC + cid
    zeros = jnp.zeros((16,), jnp.float32)

    @pl.loop(0, CHUNK)
    def _(r):
        for cc in range(0, C, 16):
            msg[r, pl.ds(cc, 16)] = zeros

    base = pl.multiple_of(sid * SEG, 8)
    nfull = SEG // CHUNK
    for t in range(nfull):
        pltpu.sync_copy(msg, aggsh.at[pl.ds(base + t * CHUNK, CHUNK)])
    rem = SEG - nfull * CHUNK
    if rem:
        pltpu.sync_copy(msg.at[pl.ds(0, rem)],
                        aggsh.at[pl.ds(base + nfull * CHUNK, rem)])
    plsc.subcore_barrier()

    kvb = (kv_a, kv_b)
    qb = (q_a, q_b)
    gsem = (sem_a, sem_b)

    def _issue(p):
        pltpu.make_async_copy(kv_hbm.at[idxb.at[p]], kvb[p % 2],
                              gsem[p % 2]).start()
        pltpu.make_async_copy(q_hbm.at[idxb.at[IBLK + p]], qb[p % 2],
                              gsem[p % 2]).start()

    def _wait(p):
        pltpu.make_async_copy(kv_hbm.at[idxb.at[p]], kvb[p % 2],
                              gsem[p % 2]).wait()
        pltpu.make_async_copy(q_hbm.at[idxb.at[IBLK + p]], qb[p % 2],
                              gsem[p % 2]).wait()

    @pl.loop(0, NBLK)
    def _(b):
        pltpu.sync_copy(rc_hbm.at[wid, b], idxb)
        _issue(0)
        for p in range(IBLK):
            if p + 1 < IBLK:
                _issue(p + 1)
            _wait(p)
            kvp, qp = kvb[p % 2], qb[p % 2]

            lanes = lax.iota(jnp.int32, 16)
            perms = [jnp.bitwise_xor(lanes, k) for k in (8, 4, 2, 1)]

            def lsum(v):
                for pm in perms:
                    v = v + v.at[pm].get(mode="promise_in_bounds")
                return v

            @pl.loop(0, CHUNK, unroll=2)
            def _(e):
                for h in range(HEADS):
                    o = h * DH
                    q = qp[e, pl.ds(o, DH)]
                    k0 = kvp[e, pl.ds(o, DH)]
                    k1 = kvp[e, pl.ds(C + o, DH)]
                    s0 = lsum(q * k0)
                    s1 = lsum(q * k1)
                    m = jnp.maximum(jnp.maximum(s0, s1), 0.0)
                    e0 = jnp.exp(s0 - m)
                    e1 = jnp.exp(s1 - m)
                    r = 1.0 / (e0 + e1 + jnp.exp(-m))
                    v0 = kvp[e, pl.ds(2 * C + o, DH)]
                    v1 = kvp[e, pl.ds(3 * C + o, DH)]
                    msg[e, pl.ds(o, DH)] = (e0 * r) * v0 + (e1 * r) * v1

            pltpu.sync_copy(msg, aggsh.at[idxb.at[IBLK + p]], add=True)

    plsc.subcore_barrier()
    pltpu.sync_copy(aggsh.at[pl.ds(base, SEG)], agg_hbm.at[cid, sid])


_BLK4 = NPAD // 8


def _combine_body(agg_ref, q_ref, kv_ref, dis_ref, g_ref, gt_ref,
                  h_ref, stats_ref):
    i = pl.program_id(0)
    agg = agg_ref[0] + agg_ref[1]
    q = q_ref[...]
    k0 = kv_ref[:, 0:C]
    k1 = kv_ref[:, C:2 * C]
    v0 = kv_ref[:, 2 * C:3 * C]
    v1 = kv_ref[:, 3 * C:4 * C]
    gmat = g_ref[...]
    gtmat = gt_ref[...]
    s0 = jnp.dot(q * k0, gmat, preferred_element_type=jnp.float32)
    s1 = jnp.dot(q * k1, gmat, preferred_element_type=jnp.float32)
    m = jnp.maximum(jnp.maximum(s0, s1), 0.0)
    e0 = jnp.exp(s0 - m)
    e1 = jnp.exp(s1 - m)
    r = 1.0 / (e0 + e1 + jnp.exp(-m))
    a0 = jnp.dot(e0 * r, gtmat, preferred_element_type=jnp.float32)
    a1 = jnp.dot(e1 * r, gtmat, preferred_element_type=jnp.float32)
    self_out = a0 * v0 + a1 * v1
    h = jnp.maximum(dis_ref[...] * (agg + self_out), 0.0)
    h_ref[...] = h
    rid = lax.broadcasted_iota(jnp.int32, (_BLK4, 1), 0) + i * _BLK4
    hm = jnp.where(rid < N, h, 0.0)

    @pl.when(i == 0)
    def _():
        stats_ref[...] = jnp.zeros_like(stats_ref)

    stats_ref[0:1, :] += jnp.sum(hm, axis=0, keepdims=True)
    stats_ref[1:2, :] += jnp.sum(hm * hm, axis=0, keepdims=True)


def _combine_call(aggp, q, kv, dis, gmat, gtmat):
    nblk = NPAD // _BLK4
    return pl.pallas_call(
        _combine_body,
        grid=(nblk,),
        in_specs=[
            pl.BlockSpec((2, _BLK4, C), lambda i: (0, i, 0)),
            pl.BlockSpec((_BLK4, C), lambda i: (i, 0)),
            pl.BlockSpec((_BLK4, 4 * C), lambda i: (i, 0)),
            pl.BlockSpec((_BLK4, 1), lambda i: (i, 0)),
            pl.BlockSpec((C, HEADS), lambda i: (0, 0)),
            pl.BlockSpec((HEADS, C), lambda i: (0, 0)),
        ],
        out_specs=[
            pl.BlockSpec((_BLK4, C), lambda i: (i, 0)),
            pl.BlockSpec((2, C), lambda i: (0, 0)),
        ],
        out_shape=[
            jax.ShapeDtypeStruct((NPAD, C), jnp.float32),
            jax.ShapeDtypeStruct((2, C), jnp.float32),
        ],
    )(aggp, q, kv, dis, gmat, gtmat)


def _bn_body(h_ref, stats_ref, gamma_ref, beta_ref, out_ref):
    inv_n = 1.0 / N
    mean = stats_ref[0:1, :] * inv_n
    var = stats_ref[1:2, :] * inv_n - mean * mean
    scale = lax.rsqrt(var + 1e-5) * gamma_ref[...]
    out_ref[...] = (h_ref[...] - mean) * scale + beta_ref[...]


def _bn_call(h, stats, gamma, beta):
    return pl.pallas_call(
        _bn_body,
        out_shape=jax.ShapeDtypeStruct((N, C), jnp.float32),
    )(h, stats, gamma, beta)


def _block_diag8(w):
    cg = C // GROUPS
    cols = []
    for g in range(GROUPS):
        col = [jnp.zeros((cg, cg), w.dtype)] * GROUPS
        col[g] = w[g]
        cols.append(jnp.concatenate(col, axis=0))
    return jnp.concatenate(cols, axis=1)


def kernel(all_node_features, edge_indices, wq, bq, wk, bk, wv, bv,
           gamma, beta):
    x = all_node_features
    row = edge_indices[0]
    col = edge_indices[1]

    pad = EPAD - E
    rowp = jnp.concatenate([row, jnp.zeros((pad,), row.dtype)])
    colp = jnp.concatenate([col, jnp.full((pad,), N, col.dtype)])
    row4 = rowp.reshape(NW, NBLK, IBLK, CHUNK)
    col4 = colp.reshape(NW, NBLK, IBLK, CHUNK)
    rc4 = jnp.concatenate([row4, col4], axis=2)
    col2 = colp.reshape(NW, DEGN, DEGC)

    wk1 = _block_diag8(wk)
    wv1 = _block_diag8(wv)
    wq1 = _block_diag8(wq)
    z = jnp.zeros((C, C), jnp.float32)
    w2k = jnp.concatenate(
        [jnp.concatenate([wk1, z], axis=1), jnp.concatenate([z, wk1], axis=1)],
        axis=0)
    w2v = jnp.concatenate(
        [jnp.concatenate([wv1, z], axis=1), jnp.concatenate([z, wv1], axis=1)],
        axis=0)
    bk2 = jnp.tile(bk, 2)[None, :]
    bv2 = jnp.tile(bv, 2)[None, :]
    bq1 = bq[None, :]

    eyeh = jnp.eye(HEADS, dtype=jnp.float32)
    gmat = jnp.repeat(eyeh, DH, axis=0)
    gtmat = jnp.repeat(eyeh, DH, axis=1)

    xp = jnp.pad(x.reshape(N, L * C), ((0, NPAD - N), (0, 0)))

    degp = _deg_kernel(col2)
    degp2 = degp.reshape(NC, NPAD).T
    kv, q, dis = _tables_call(xp, w2k, w2v, wq1, bk2, bv2, bq1, degp2)
    aggp = _edge_kernel(rc4, kv, q)
    aggp = aggp.reshape(NC, NPAD, C)
    h, stats = _combine_call(aggp, q, kv, dis, gmat, gtmat)
    return _bn_call(h[:N], stats, gamma[None, :], beta[None, :])


# speedup vs baseline: 3.6782x; 2.5634x over previous
import dataclasses
import functools

import jax
import jax.numpy as jnp
from jax import lax
from jax.experimental import pallas as pl
from jax.experimental.pallas import tpu as pltpu
from jax.experimental.pallas import tpu_sc as plsc

N = 10000
E = 160000
C = 128
L = 2
HEADS = 8
DH = C // HEADS
GROUPS = 8

NC = 2
NS = 16
NW = NC * NS
NPAD = 10112
SEG = NPAD // NS
CHUNK = 32
EPT = 5120
EPAD = NW * EPT
NCHUNK = EPT // CHUNK
IBLK = 8
NBLK = NCHUNK // IBLK

_mesh = plsc.VectorSubcoreMesh(core_axis_name="c", subcore_axis_name="s")
_cp = pltpu.CompilerParams()
if "needs_layout_passes" in pltpu.CompilerParams.__dataclass_fields__:
    _cp = dataclasses.replace(_cp, needs_layout_passes=False)

DEGC = 128
DEGN = EPT // DEGC


@functools.partial(
    pl.kernel,
    out_type=jax.ShapeDtypeStruct((NC, NS, SEG), jnp.float32),
    mesh=_mesh,
    compiler_params=_cp,
    scratch_types=[
        pltpu.VMEM((DEGC,), jnp.float32),
        pltpu.VMEM((SEG,), jnp.float32),
        pltpu.VMEM((DEGN, DEGC), jnp.int32),
        pltpu.VMEM_SHARED((NPAD,), jnp.float32),
    ],
)
def _deg_kernel(col_hbm, deg_hbm, ones, acc, idx, sh):
    cid = lax.axis_index("c")
    sid = lax.axis_index("s")
    wid = sid * NC + cid
    one16 = jnp.full((16,), 1.0, jnp.float32)
    zero16 = jnp.zeros((16,), jnp.float32)

    @pl.loop(0, DEGC, step=16)
    def _(i):
        ones[pl.ds(i, 16)] = one16

    @pl.loop(0, SEG, step=16)
    def _(i):
        acc[pl.ds(i, 16)] = zero16

    base = pl.multiple_of(sid * SEG, 8)
    pltpu.sync_copy(acc, sh.at[pl.ds(base, SEG)])
    plsc.subcore_barrier()

    pltpu.sync_copy(col_hbm.at[wid], idx)

    @pl.loop(0, DEGN)
    def _(t):
        pltpu.sync_copy(ones, sh.at[idx.at[t]], add=True)

    plsc.subcore_barrier()
    pltpu.sync_copy(sh.at[pl.ds(base, SEG)], acc)
    pltpu.sync_copy(acc, deg_hbm.at[cid, sid])


_BLK2 = NPAD // 8


def _tables_body(x_ref, w2k_ref, w2v_ref, wq_ref, bk2_ref, bv2_ref, bq_ref,
                 degp_ref, kv_ref, q_ref, dis_ref):
    deg = degp_ref[:, 0:1] + degp_ref[:, 1:2] + 1.0
    dis = lax.rsqrt(deg)
    x = x_ref[...]
    k = jnp.dot(x, w2k_ref[...], preferred_element_type=jnp.float32)
    k = k + bk2_ref[...]
    v = jnp.dot(x, w2v_ref[...], preferred_element_type=jnp.float32)
    v = (v + bv2_ref[...]) * dis
    q = jnp.dot(x[:, C:], wq_ref[...], preferred_element_type=jnp.float32)
    q_ref[...] = (q + bq_ref[...]) * 0.25
    kv_ref[...] = jnp.concatenate([k, v], axis=1)
    dis_ref[...] = dis


def _tables_call(xp, w2k, w2v, wq1, bk2, bv2, bq1, degp2):
    nblk = NPAD // _BLK2
    return pl.pallas_call(
        _tables_body,
        grid=(nblk,),
        in_specs=[
            pl.BlockSpec((_BLK2, 2 * C), lambda i: (i, 0)),
            pl.BlockSpec((2 * C, 2 * C), lambda i: (0, 0)),
            pl.BlockSpec((2 * C, 2 * C), lambda i: (0, 0)),
            pl.BlockSpec((C, C), lambda i: (0, 0)),
            pl.BlockSpec((1, 2 * C), lambda i: (0, 0)),
            pl.BlockSpec((1, 2 * C), lambda i: (0, 0)),
            pl.BlockSpec((1, C), lambda i: (0, 0)),
            pl.BlockSpec((_BLK2, 2), lambda i: (i, 0)),
        ],
        out_specs=[
            pl.BlockSpec((_BLK2, 4 * C), lambda i: (i, 0)),
            pl.BlockSpec((_BLK2, C), lambda i: (i, 0)),
            pl.BlockSpec((_BLK2, 1), lambda i: (i, 0)),
        ],
        out_shape=[
            jax.ShapeDtypeStruct((NPAD, 4 * C), jnp.float32),
            jax.ShapeDtypeStruct((NPAD, C), jnp.float32),
            jax.ShapeDtypeStruct((NPAD, 1), jnp.float32),
        ],
    )(xp, w2k, w2v, wq1, bk2, bv2, bq1, degp2)


@functools.partial(
    pl.kernel,
    out_type=jax.ShapeDtypeStruct((NC, NS, SEG, C), jnp.float32),
    mesh=_mesh,
    compiler_params=_cp,
    scratch_types=[
        pltpu.VMEM((2 * IBLK, CHUNK), jnp.int32),
        pltpu.VMEM((CHUNK, 4 * C), jnp.float32),
        pltpu.VMEM((CHUNK, 4 * C), jnp.float32),
        pltpu.VMEM((CHUNK, C), jnp.float32),
        pltpu.VMEM((CHUNK, C), jnp.float32),
        pltpu.VMEM((CHUNK, C), jnp.float32),
        pltpu.VMEM_SHARED((NPAD, C), jnp.float32),
        pltpu.SemaphoreType.DMA,
        pltpu.SemaphoreType.DMA,
        pltpu.SemaphoreType.DMA,
    ],
)
def _edge_kernel(rc_hbm, kv_hbm, q_hbm, agg_hbm,
                 idxb, kv_a, kv_b, q_a, q_b, msg,
                 aggsh, sem_a, sem_b, sem_s):
    cid = lax.axis_index("c")
    sid = lax.axis_index("s")
    wid = sid * NC + cid
    zeros = jnp.zeros((16,), jnp.float32)

    @pl.loop(0, CHUNK)
    def _(r):
        for cc in range(0, C, 16):
            msg[r, pl.ds(cc, 16)] = zeros

    base = pl.multiple_of(sid * SEG, 8)
    nfull = SEG // CHUNK
    for t in range(nfull):
        pltpu.sync_copy(msg, aggsh.at[pl.ds(base + t * CHUNK, CHUNK)])
    rem = SEG - nfull * CHUNK
    if rem:
        pltpu.sync_copy(msg.at[pl.ds(0, rem)],
                        aggsh.at[pl.ds(base + nfull * CHUNK, rem)])
    plsc.subcore_barrier()

    kvb = (kv_a, kv_b)
    qb = (q_a, q_b)
    gsem = (sem_a, sem_b)

    def _issue(p):
        pltpu.make_async_copy(kv_hbm.at[idxb.at[p]], kvb[p % 2],
                              gsem[p % 2]).start()
        pltpu.make_async_copy(q_hbm.at[idxb.at[IBLK + p]], qb[p % 2],
                              gsem[p % 2]).start()

    def _wait(p):
        pltpu.make_async_copy(kv_hbm.at[idxb.at[p]], kvb[p % 2],
                              gsem[p % 2]).wait()
        pltpu.make_async_copy(q_hbm.at[idxb.at[IBLK + p]], qb[p % 2],
                              gsem[p % 2]).wait()

    @pl.loop(0, NBLK)
    def _(b):
        pltpu.sync_copy(rc_hbm.at[wid, b], idxb)
        _issue(0)
        for p in range(IBLK):
            if p + 1 < IBLK:
                _issue(p + 1)
            _wait(p)
            kvp, qp = kvb[p % 2], qb[p % 2]

            lanes = lax.iota(jnp.int32, 16)
            perms = [jnp.bitwise_xor(lanes, k) for k in (8, 4, 2, 1)]

            def lsum(v):
                for pm in perms:
                    v = v + v.at[pm].get(mode="promise_in_bounds")
                return v

            hs = range(HEADS)

            @pl.loop(0, CHUNK)
            def _(e):
                qs = [qp[e, pl.ds(h * DH, DH)] for h in hs]
                k0s = [kvp[e, pl.ds(h * DH, DH)] for h in hs]
                k1s = [kvp[e, pl.ds(C + h * DH, DH)] for h in hs]
                ps = ([qs[h] * k0s[h] for h in hs]
                      + [qs[h] * k1s[h] for h in hs])
                for pm in perms:
                    ps = [v + v.at[pm].get(mode="promise_in_bounds")
                          for v in ps]
                s0s, s1s = ps[:HEADS], ps[HEADS:]
                ms = [jnp.maximum(jnp.maximum(s0s[h], s1s[h]), 0.0)
                      for h in hs]
                e0s = [jnp.exp(s0s[h] - ms[h]) for h in hs]
                e1s = [jnp.exp(s1s[h] - ms[h]) for h in hs]
                ens = [jnp.exp(-ms[h]) for h in hs]
                rs = [1.0 / (e0s[h] + e1s[h] + ens[h]) for h in hs]
                v0s = [kvp[e, pl.ds(2 * C + h * DH, DH)] for h in hs]
                v1s = [kvp[e, pl.ds(3 * C + h * DH, DH)] for h in hs]
                for h in hs:
                    msg[e, pl.ds(h * DH, DH)] = ((e0s[h] * rs[h]) * v0s[h]
                                                 + (e1s[h] * rs[h]) * v1s[h])

            pltpu.sync_copy(msg, aggsh.at[idxb.at[IBLK + p]], add=True)

    plsc.subcore_barrier()
    pltpu.sync_copy(aggsh.at[pl.ds(base, SEG)], agg_hbm.at[cid, sid])


_BLK4 = NPAD // 8


def _combine_body(agg_ref, q_ref, kv_ref, dis_ref, g_ref, gt_ref,
                  h_ref, stats_ref):
    i = pl.program_id(0)
    agg = agg_ref[0] + agg_ref[1]
    q = q_ref[...]
    k0 = kv_ref[:, 0:C]
    k1 = kv_ref[:, C:2 * C]
    v0 = kv_ref[:, 2 * C:3 * C]
    v1 = kv_ref[:, 3 * C:4 * C]
    gmat = g_ref[...]
    gtmat = gt_ref[...]
    s0 = jnp.dot(q * k0, gmat, preferred_element_type=jnp.float32)
    s1 = jnp.dot(q * k1, gmat, preferred_element_type=jnp.float32)
    m = jnp.maximum(jnp.maximum(s0, s1), 0.0)
    e0 = jnp.exp(s0 - m)
    e1 = jnp.exp(s1 - m)
    r = 1.0 / (e0 + e1 + jnp.exp(-m))
    a0 = jnp.dot(e0 * r, gtmat, preferred_element_type=jnp.float32)
    a1 = jnp.dot(e1 * r, gtmat, preferred_element_type=jnp.float32)
    self_out = a0 * v0 + a1 * v1
    h = jnp.maximum(dis_ref[...] * (agg + self_out), 0.0)
    h_ref[...] = h
    rid = lax.broadcasted_iota(jnp.int32, (_BLK4, 1), 0) + i * _BLK4
    hm = jnp.where(rid < N, h, 0.0)

    @pl.when(i == 0)
    def _():
        stats_ref[...] = jnp.zeros_like(stats_ref)

    stats_ref[0:1, :] += jnp.sum(hm, axis=0, keepdims=True)
    stats_ref[1:2, :] += jnp.sum(hm * hm, axis=0, keepdims=True)


def _combine_call(aggp, q, kv, dis, gmat, gtmat):
    nblk = NPAD // _BLK4
    return pl.pallas_call(
        _combine_body,
        grid=(nblk,),
        in_specs=[
            pl.BlockSpec((2, _BLK4, C), lambda i: (0, i, 0)),
            pl.BlockSpec((_BLK4, C), lambda i: (i, 0)),
            pl.BlockSpec((_BLK4, 4 * C), lambda i: (i, 0)),
            pl.BlockSpec((_BLK4, 1), lambda i: (i, 0)),
            pl.BlockSpec((C, HEADS), lambda i: (0, 0)),
            pl.BlockSpec((HEADS, C), lambda i: (0, 0)),
        ],
        out_specs=[
            pl.BlockSpec((_BLK4, C), lambda i: (i, 0)),
            pl.BlockSpec((2, C), lambda i: (0, 0)),
        ],
        out_shape=[
            jax.ShapeDtypeStruct((NPAD, C), jnp.float32),
            jax.ShapeDtypeStruct((2, C), jnp.float32),
        ],
    )(aggp, q, kv, dis, gmat, gtmat)


def _bn_body(h_ref, stats_ref, gamma_ref, beta_ref, out_ref):
    inv_n = 1.0 / N
    mean = stats_ref[0:1, :] * inv_n
    var = stats_ref[1:2, :] * inv_n - mean * mean
    scale = lax.rsqrt(var + 1e-5) * gamma_ref[...]
    out_ref[...] = (h_ref[...] - mean) * scale + beta_ref[...]


def _bn_call(h, stats, gamma, beta):
    return pl.pallas_call(
        _bn_body,
        out_shape=jax.ShapeDtypeStruct((N, C), jnp.float32),
    )(h, stats, gamma, beta)


def _block_diag8(w):
    cg = C // GROUPS
    cols = []
    for g in range(GROUPS):
        col = [jnp.zeros((cg, cg), w.dtype)] * GROUPS
        col[g] = w[g]
        cols.append(jnp.concatenate(col, axis=0))
    return jnp.concatenate(cols, axis=1)


def kernel(all_node_features, edge_indices, wq, bq, wk, bk, wv, bv,
           gamma, beta):
    x = all_node_features
    row = edge_indices[0]
    col = edge_indices[1]

    pad = EPAD - E
    rowp = jnp.concatenate([row, jnp.zeros((pad,), row.dtype)])
    colp = jnp.concatenate([col, jnp.full((pad,), N, col.dtype)])
    row4 = rowp.reshape(NW, NBLK, IBLK, CHUNK)
    col4 = colp.reshape(NW, NBLK, IBLK, CHUNK)
    rc4 = jnp.concatenate([row4, col4], axis=2)
    col2 = colp.reshape(NW, DEGN, DEGC)

    wk1 = _block_diag8(wk)
    wv1 = _block_diag8(wv)
    wq1 = _block_diag8(wq)
    z = jnp.zeros((C, C), jnp.float32)
    w2k = jnp.concatenate(
        [jnp.concatenate([wk1, z], axis=1), jnp.concatenate([z, wk1], axis=1)],
        axis=0)
    w2v = jnp.concatenate(
        [jnp.concatenate([wv1, z], axis=1), jnp.concatenate([z, wv1], axis=1)],
        axis=0)
    bk2 = jnp.tile(bk, 2)[None, :]
    bv2 = jnp.tile(bv, 2)[None, :]
    bq1 = bq[None, :]

    eyeh = jnp.eye(HEADS, dtype=jnp.float32)
    gmat = jnp.repeat(eyeh, DH, axis=0)
    gtmat = jnp.repeat(eyeh, DH, axis=1)

    xp = jnp.pad(x.reshape(N, L * C), ((0, NPAD - N), (0, 0)))

    degp = _deg_kernel(col2)
    degp2 = degp.reshape(NC, NPAD).T
    kv, q, dis = _tables_call(xp, w2k, w2v, wq1, bk2, bv2, bq1, degp2)
    aggp = _edge_kernel(rc4, kv, q)
    aggp = aggp.reshape(NC, NPAD, C)
    h, stats = _combine_call(aggp, q, kv, dis, gmat, gtmat)
    return _bn_call(h[:N], stats, gamma[None, :], beta[None, :])
